# Initial kernel scaffold; baseline (speedup 1.0000x reference)
import jax
import jax.numpy as jnp
from jax.experimental import pallas as pl


def kernel(x, edge_index, W1, filt1, b1, W2, filt2, b2, d_rows, d_cols, d_vals):
    raise NotImplementedError("write your pallas kernel here")


# speedup vs baseline: 8.3479x; 8.3479x over previous
import functools

import jax
import jax.numpy as jnp
from jax import lax
from jax.experimental import pallas as pl
from jax.experimental.pallas import tpu as pltpu
from jax.experimental.pallas import tpu_sc as plsc

N = 10000
NPAD = 10240
IN_F = 128
HID = 64
NC = 40
NCP = 64
RL = 4
NK = 3
NNZ = 160000
NTILES = 16
EDG_T = NNZ // NTILES
CH = 80
NCHUNK = EDG_T // CH
RPT = NPAD // NTILES
FH = 32
THRESH = 1e-4


def _bcast16(vec, i):
    return vec.at[jnp.full((16,), i, jnp.int32)].get(mode="promise_in_bounds")


def _scale_chunk(g_v, vals_v, j):
    for g in range(CH // 16):
        vv = vals_v[j, pl.ds(g * 16, 16)]
        for i in range(16):
            r = g * 16 + i
            fb = _bcast16(vv, i)
            g_v[r, pl.ds(0, 16)] = g_v[r, pl.ds(0, 16)] * fb
            g_v[r, pl.ds(16, 16)] = g_v[r, pl.ds(16, 16)] * fb


def _sc_layer(h, dr, dc, dv, filt):
    mesh = plsc.VectorSubcoreMesh(core_axis_name="c", subcore_axis_name="s")

    @functools.partial(
        pl.kernel,
        mesh=mesh,
        compiler_params=pltpu.CompilerParams(use_tc_tiling_on_sc=False),
        out_type=jax.ShapeDtypeStruct((2, NPAD, FH), jnp.float32),
        scratch_types=[
            pltpu.VMEM((NCHUNK, CH), jnp.int32),
            pltpu.VMEM((NCHUNK, CH), jnp.int32),
            pltpu.VMEM((NCHUNK, CH), jnp.float32),
            pltpu.VMEM((CH, FH), jnp.float32),
            pltpu.VMEM((RPT, FH), jnp.float32),
            pltpu.VMEM((RPT,), jnp.float32),
            pltpu.VMEM_SHARED((NPAD, FH), jnp.float32),
            pltpu.VMEM_SHARED((NPAD, FH), jnp.float32),
            pltpu.VMEM_SHARED((NPAD, FH), jnp.float32),
            pltpu.SemaphoreType.DMA,
        ],
    )
    def k(h_hbm, dr_hbm, dc_hbm, dv_hbm, f_hbm, out_hbm,
          cols_v, rows_v, vals_v, g_v, s_v, f_v, h_s, t_s, o_s, sem):
        cid = lax.axis_index("c")
        sid = lax.axis_index("s")
        base = sid * RPT
        zrow = jnp.zeros((16,), jnp.float32)

        def zero_body(i, _):
            s_v[i, pl.ds(0, 16)] = zrow
            s_v[i, pl.ds(16, 16)] = zrow
            return 0

        lax.fori_loop(0, RPT, zero_body, 0)
        pltpu.sync_copy(s_v, o_s.at[pl.ds(base, RPT)])
        pltpu.sync_copy(h_hbm.at[cid, pl.ds(base, RPT)], s_v)
        pltpu.sync_copy(s_v, h_s.at[pl.ds(base, RPT)])
        plsc.subcore_barrier()

        for k_i in range(NK):
            lax.fori_loop(0, RPT, zero_body, 0)
            pltpu.sync_copy(s_v, t_s.at[pl.ds(base, RPT)])
            pltpu.sync_copy(dc_hbm.at[k_i, sid], cols_v)
            pltpu.sync_copy(dr_hbm.at[k_i, sid], rows_v)
            pltpu.sync_copy(dv_hbm.at[k_i, sid], vals_v)
            plsc.subcore_barrier()

            def dec_body(j, _):
                pltpu.async_copy(h_s.at[cols_v.at[j]], g_v, sem).wait()
                _scale_chunk(g_v, vals_v, j)
                pltpu.sync_copy(g_v, t_s.at[rows_v.at[j]], add=True)
                return 0

            lax.fori_loop(0, NCHUNK, dec_body, 0)
            plsc.subcore_barrier()

            pltpu.sync_copy(t_s.at[pl.ds(base, RPT)], s_v)
            pltpu.sync_copy(f_hbm.at[k_i, sid], f_v)

            def shr_body(g, _):
                fv = f_v[pl.ds(g * 16, 16)]
                for i in range(16):
                    r = g * 16 + i
                    fb = _bcast16(fv, i)
                    for half in range(2):
                        sl = pl.ds(half * 16, 16)
                        v = s_v[r, sl]
                        a = jnp.maximum(jnp.abs(v) - THRESH, 0.0)
                        s_v[r, sl] = jnp.sign(v) * a * fb
                return 0

            lax.fori_loop(0, RPT // 16, shr_body, 0)
            pltpu.sync_copy(s_v, t_s.at[pl.ds(base, RPT)])
            plsc.subcore_barrier()

            def rec_body(j, _):
                pltpu.async_copy(t_s.at[cols_v.at[j]], g_v, sem).wait()
                _scale_chunk(g_v, vals_v, j)
                pltpu.sync_copy(g_v, o_s.at[rows_v.at[j]], add=True)
                return 0

            lax.fori_loop(0, NCHUNK, rec_body, 0)
            plsc.subcore_barrier()

        pltpu.sync_copy(o_s.at[pl.ds(base, RPT)], s_v)
        pltpu.sync_copy(s_v, out_hbm.at[cid, pl.ds(base, RPT)])

    return k(h, dr, dc, dv, filt)


BN = 1280


def _mm1_body(x_ref, w_ref, o_ref):
    h = jnp.dot(x_ref[...], w_ref[...], preferred_element_type=jnp.float32)
    o_ref[0] = h[:, :FH]
    o_ref[1] = h[:, FH:]


def _mm1(x, w):
    return pl.pallas_call(
        _mm1_body,
        grid=(NPAD // BN,),
        in_specs=[
            pl.BlockSpec((BN, IN_F), lambda i: (i, 0)),
            pl.BlockSpec((IN_F, HID), lambda i: (0, 0)),
        ],
        out_specs=pl.BlockSpec((2, BN, FH), lambda i: (0, i, 0)),
        out_shape=jax.ShapeDtypeStruct((2, NPAD, FH), jnp.float32),
    )(x, w)


def _mm2_body(h_ref, w_ref, b_ref, o_ref):
    acc = jnp.dot(h_ref[0], w_ref[:FH, :], preferred_element_type=jnp.float32)
    acc += jnp.dot(h_ref[1], w_ref[FH:, :], preferred_element_type=jnp.float32)
    acc += b_ref[...]
    o_ref[0] = acc[:, :FH]
    o_ref[1] = acc[:, FH:]


def _mm2(h, w, brow):
    return pl.pallas_call(
        _mm2_body,
        grid=(NPAD // BN,),
        in_specs=[
            pl.BlockSpec((2, BN, FH), lambda i: (0, i, 0)),
            pl.BlockSpec((HID, NCP), lambda i: (0, 0)),
            pl.BlockSpec((1, NCP), lambda i: (0, 0)),
        ],
        out_specs=pl.BlockSpec((2, BN, FH), lambda i: (0, i, 0)),
        out_shape=jax.ShapeDtypeStruct((2, NPAD, FH), jnp.float32),
    )(h, w, brow)


FBN = 2000


def _final_body(h_ref, b_ref, o_ref):
    y = jnp.concatenate([h_ref[0], h_ref[1]], axis=1) + b_ref[...]
    col = lax.broadcasted_iota(jnp.int32, (FBN, NCP), 1)
    valid = col < NC
    neg = jnp.full_like(y, -jnp.inf)
    m = jnp.max(jnp.where(valid, y, neg), axis=1, keepdims=True)
    e = jnp.where(valid, jnp.exp(y - m), 0.0)
    s = jnp.sum(e, axis=1, keepdims=True)
    o_ref[...] = (y - m - jnp.log(s))[:, :NC]


def _final(h, b2p):
    return pl.pallas_call(
        _final_body,
        grid=(N // FBN,),
        in_specs=[
            pl.BlockSpec((2, FBN, FH), lambda i: (0, i, 0)),
            pl.BlockSpec((1, NCP), lambda i: (0, 0)),
        ],
        out_specs=pl.BlockSpec((FBN, NC), lambda i: (i, 0)),
        out_shape=jax.ShapeDtypeStruct((N, NC), jnp.float32),
    )(h, b2p)


def kernel(x, edge_index, W1, filt1, b1, W2, filt2, b2, d_rows, d_cols, d_vals):
    dr = d_rows[1:RL].reshape(NK, NTILES, NCHUNK, CH)
    dc = d_cols[1:RL].reshape(NK, NTILES, NCHUNK, CH)
    dv = d_vals[1:RL].reshape(NK, NTILES, NCHUNK, CH)
    f1 = jnp.pad(filt1.reshape(RL, N)[1:RL], ((0, 0), (0, NPAD - N)))
    f1 = f1.reshape(NK, NTILES, RPT)
    f2 = jnp.pad(filt2.reshape(RL, N)[1:RL], ((0, 0), (0, NPAD - N)))
    f2 = f2.reshape(NK, NTILES, RPT)
    xp = jnp.pad(x, ((0, NPAD - N), (0, 0)))
    W2p = jnp.pad(W2, ((0, 0), (0, NCP - NC)))
    bias2 = (b1 @ W2p).reshape(1, NCP)
    b2p = jnp.pad(b2, (0, NCP - NC)).reshape(1, NCP)

    h1 = _mm1(xp, W1)
    o1 = _sc_layer(h1, dr, dc, dv, f1)
    h2 = _mm2(o1, W2p, bias2)
    o2 = _sc_layer(h2, dr, dc, dv, f2)
    o2c = lax.slice(o2, (0, 0, 0), (2, N, FH))
    return _final(o2c, b2p)


# speedup vs baseline: 11.9225x; 1.4282x over previous
import functools

import jax
import jax.numpy as jnp
from jax import lax
from jax.experimental import pallas as pl
from jax.experimental.pallas import tpu as pltpu
from jax.experimental.pallas import tpu_sc as plsc

N = 10000
NPAD = 10240
IN_F = 128
HID = 64
NC = 40
NCP = 64
RL = 4
NK = 3
NNZ = 160000
NTILES = 16
EDG_T = NNZ // NTILES
CH = 80
EDG_TP = 10240
NCH = EDG_TP // CH
NBUF = 4
RPT = NPAD // NTILES
HSTG = RPT // 2
FH = 32
THRESH = 1e-4


def _bcast16(vec, i):
    return vec.at[jnp.full((16,), i, jnp.int32)].get(mode="promise_in_bounds")


def _scale_to(dst, src, vals_v, j):
    for g in range(CH // 16):
        vv = vals_v[j, pl.ds(g * 16, 16)]
        for i in range(16):
            r = g * 16 + i
            fb = _bcast16(vv, i)
            dst[r, pl.ds(0, 16)] = src[r, pl.ds(0, 16)] * fb
            dst[r, pl.ds(16, 16)] = src[r, pl.ds(16, 16)] * fb


def _sc_layer(h, dr, dc, dv, filt):
    mesh = plsc.VectorSubcoreMesh(core_axis_name="c", subcore_axis_name="s")

    @functools.partial(
        pl.kernel,
        mesh=mesh,
        compiler_params=pltpu.CompilerParams(use_tc_tiling_on_sc=False),
        out_type=jax.ShapeDtypeStruct((2, NPAD, FH), jnp.float32),
        scratch_types=[
            pltpu.VMEM((NCH, CH), jnp.int32),
            pltpu.VMEM((NCH, CH), jnp.int32),
            pltpu.VMEM((NCH, CH), jnp.float32),
            [pltpu.VMEM((CH, FH), jnp.float32) for _ in range(NBUF)],
            [pltpu.VMEM((CH, FH), jnp.float32) for _ in range(NBUF)],
            pltpu.VMEM((HSTG, FH), jnp.float32),
            pltpu.VMEM((RPT,), jnp.float32),
            pltpu.VMEM_SHARED((NPAD, FH), jnp.float32),
            pltpu.VMEM_SHARED((NPAD, FH), jnp.float32),
            pltpu.VMEM_SHARED((NPAD, FH), jnp.float32),
            pltpu.SemaphoreType.DMA((NBUF,)),
            pltpu.SemaphoreType.DMA((NBUF,)),
        ],
    )
    def k(h_hbm, dr_hbm, dc_hbm, dv_hbm, f_hbm, out_hbm,
          cols_v, rows_v, vals_v, g_b, s_b, stg_v, f_v, h_s, t_s, o_s,
          gsem, ssem):
        cid = lax.axis_index("c")
        sid = lax.axis_index("s")
        base = sid * RPT
        zrow = jnp.zeros((16,), jnp.float32)

        def zero_stg(i, _):
            stg_v[i, pl.ds(0, 16)] = zrow
            stg_v[i, pl.ds(16, 16)] = zrow
            return 0

        def zero_rings(i, _):
            for b in range(NBUF):
                s_b[b][i, pl.ds(0, 16)] = zrow
                s_b[b][i, pl.ds(16, 16)] = zrow
            return 0

        def phase(table, acc):
            lax.fori_loop(0, CH, zero_rings, 0)
            for b in range(NBUF):
                pltpu.async_copy(table.at[cols_v.at[b]], g_b[b], gsem.at[b])
                pltpu.async_copy(s_b[b], acc.at[rows_v.at[b]], ssem.at[b],
                                 add=True)

            def ring(gi, _):
                for b in range(NBUF):
                    j = gi * NBUF + b
                    pltpu.make_async_copy(
                        table.at[cols_v.at[j]], g_b[b], gsem.at[b]).wait()
                    pltpu.make_async_copy(
                        s_b[b], acc.at[rows_v.at[j]], ssem.at[b]).wait()
                    _scale_to(s_b[b], g_b[b], vals_v, j)
                    jn = jnp.minimum(j + NBUF, NCH - 1)
                    pltpu.async_copy(table.at[cols_v.at[jn]], g_b[b],
                                     gsem.at[b])
                    pltpu.async_copy(s_b[b], acc.at[rows_v.at[j]],
                                     ssem.at[b], add=True)
                return 0

            lax.fori_loop(0, NCH // NBUF, ring, 0)
            for b in range(NBUF):
                pltpu.make_async_copy(
                    table.at[cols_v.at[0]], g_b[b], gsem.at[b]).wait()
                pltpu.make_async_copy(
                    s_b[b], acc.at[rows_v.at[0]], ssem.at[b]).wait()

        lax.fori_loop(0, HSTG, zero_stg, 0)
        for hh in range(2):
            sl = pl.ds(base + hh * HSTG, HSTG)
            pltpu.sync_copy(stg_v, o_s.at[sl])
        for hh in range(2):
            sl = pl.ds(base + hh * HSTG, HSTG)
            pltpu.sync_copy(h_hbm.at[cid, sl], stg_v)
            pltpu.sync_copy(stg_v, h_s.at[sl])
        plsc.subcore_barrier()

        def k_body(k_i, _):
            lax.fori_loop(0, HSTG, zero_stg, 0)
            for hh in range(2):
                pltpu.sync_copy(stg_v, t_s.at[pl.ds(base + hh * HSTG, HSTG)])
            pltpu.sync_copy(dc_hbm.at[k_i, sid], cols_v)
            pltpu.sync_copy(dr_hbm.at[k_i, sid], rows_v)
            pltpu.sync_copy(dv_hbm.at[k_i, sid], vals_v)
            pltpu.sync_copy(f_hbm.at[k_i, sid], f_v)
            plsc.subcore_barrier()

            phase(h_s, t_s)
            plsc.subcore_barrier()

            for hh in range(2):
                sl = pl.ds(base + hh * HSTG, HSTG)
                pltpu.sync_copy(t_s.at[sl], stg_v)

                def shr_body(g, _):
                    fv = f_v[pl.ds(hh * HSTG + g * 16, 16)]
                    for i in range(16):
                        r = g * 16 + i
                        fb = _bcast16(fv, i)
                        for half in range(2):
                            fsl = pl.ds(half * 16, 16)
                            v = stg_v[r, fsl]
                            a = jnp.maximum(jnp.abs(v) - THRESH, 0.0)
                            stg_v[r, fsl] = jnp.sign(v) * a * fb
                    return 0

                lax.fori_loop(0, HSTG // 16, shr_body, 0)
                pltpu.sync_copy(stg_v, t_s.at[sl])
            plsc.subcore_barrier()

            phase(t_s, o_s)
            plsc.subcore_barrier()
            return 0

        lax.fori_loop(0, NK, k_body, 0)

        for hh in range(2):
            sl = pl.ds(base + hh * HSTG, HSTG)
            pltpu.sync_copy(o_s.at[sl], stg_v)
            pltpu.sync_copy(stg_v, out_hbm.at[cid, sl])

    return k(h, dr, dc, dv, filt)


BN = 1280


def _mm1_body(x_ref, w_ref, o_ref):
    h = jnp.dot(x_ref[...], w_ref[...], preferred_element_type=jnp.float32)
    o_ref[0] = h[:, :FH]
    o_ref[1] = h[:, FH:]


def _mm1(x, w):
    return pl.pallas_call(
        _mm1_body,
        grid=(NPAD // BN,),
        in_specs=[
            pl.BlockSpec((BN, IN_F), lambda i: (i, 0)),
            pl.BlockSpec((IN_F, HID), lambda i: (0, 0)),
        ],
        out_specs=pl.BlockSpec((2, BN, FH), lambda i: (0, i, 0)),
        out_shape=jax.ShapeDtypeStruct((2, NPAD, FH), jnp.float32),
    )(x, w)


def _mm2_body(h_ref, w_ref, b_ref, o_ref):
    acc = jnp.dot(h_ref[0], w_ref[:FH, :], preferred_element_type=jnp.float32)
    acc += jnp.dot(h_ref[1], w_ref[FH:, :], preferred_element_type=jnp.float32)
    acc += b_ref[...]
    o_ref[0] = acc[:, :FH]
    o_ref[1] = acc[:, FH:]


def _mm2(h, w, brow):
    return pl.pallas_call(
        _mm2_body,
        grid=(NPAD // BN,),
        in_specs=[
            pl.BlockSpec((2, BN, FH), lambda i: (0, i, 0)),
            pl.BlockSpec((HID, NCP), lambda i: (0, 0)),
            pl.BlockSpec((1, NCP), lambda i: (0, 0)),
        ],
        out_specs=pl.BlockSpec((2, BN, FH), lambda i: (0, i, 0)),
        out_shape=jax.ShapeDtypeStruct((2, NPAD, FH), jnp.float32),
    )(h, w, brow)


FBN = 2000


def _final_body(h_ref, b_ref, o_ref):
    y = jnp.concatenate([h_ref[0], h_ref[1]], axis=1) + b_ref[...]
    col = lax.broadcasted_iota(jnp.int32, (FBN, NCP), 1)
    valid = col < NC
    neg = jnp.full_like(y, -jnp.inf)
    m = jnp.max(jnp.where(valid, y, neg), axis=1, keepdims=True)
    e = jnp.where(valid, jnp.exp(y - m), 0.0)
    s = jnp.sum(e, axis=1, keepdims=True)
    o_ref[...] = (y - m - jnp.log(s))[:, :NC]


def _final(h, b2p):
    return pl.pallas_call(
        _final_body,
        grid=(N // FBN,),
        in_specs=[
            pl.BlockSpec((2, FBN, FH), lambda i: (0, i, 0)),
            pl.BlockSpec((1, NCP), lambda i: (0, 0)),
        ],
        out_specs=pl.BlockSpec((FBN, NC), lambda i: (i, 0)),
        out_shape=jax.ShapeDtypeStruct((N, NC), jnp.float32),
    )(h, b2p)


def kernel(x, edge_index, W1, filt1, b1, W2, filt2, b2, d_rows, d_cols, d_vals):
    pad = ((0, 0), (0, 0), (0, EDG_TP - EDG_T))
    dr = jnp.pad(d_rows[1:RL].reshape(NK, NTILES, EDG_T), pad)
    dr = dr.reshape(NK, NTILES, NCH, CH)
    dc = jnp.pad(d_cols[1:RL].reshape(NK, NTILES, EDG_T), pad)
    dc = dc.reshape(NK, NTILES, NCH, CH)
    dv = jnp.pad(d_vals[1:RL].reshape(NK, NTILES, EDG_T), pad)
    dv = dv.reshape(NK, NTILES, NCH, CH)
    f1 = jnp.pad(filt1.reshape(RL, N)[1:RL], ((0, 0), (0, NPAD - N)))
    f1 = f1.reshape(NK, NTILES, RPT)
    f2 = jnp.pad(filt2.reshape(RL, N)[1:RL], ((0, 0), (0, NPAD - N)))
    f2 = f2.reshape(NK, NTILES, RPT)
    xp = jnp.pad(x, ((0, NPAD - N), (0, 0)))
    W2p = jnp.pad(W2, ((0, 0), (0, NCP - NC)))
    bias2 = (b1 @ W2p).reshape(1, NCP)
    b2p = jnp.pad(b2, (0, NCP - NC)).reshape(1, NCP)

    h1 = _mm1(xp, W1)
    o1 = _sc_layer(h1, dr, dc, dv, f1)
    h2 = _mm2(o1, W2p, bias2)
    o2 = _sc_layer(h2, dr, dc, dv, f2)
    o2c = lax.slice(o2, (0, 0, 0), (2, N, FH))
    return _final(o2c, b2p)


# speedup vs baseline: 13.8156x; 1.1588x over previous
import functools

import jax
import jax.numpy as jnp
from jax import lax
from jax.experimental import pallas as pl
from jax.experimental.pallas import tpu as pltpu
from jax.experimental.pallas import tpu_sc as plsc

N = 10000
NPAD = 10240
IN_F = 128
HID = 64
NC = 40
NCP = 64
RL = 4
NK = 3
NNZ = 160000
NTILES = 16
EDG_T = NNZ // NTILES
CH = 80
EDG_TP = 10240
NCH = EDG_TP // CH
NBUF = 4
RPT = NPAD // NTILES
HSTG = RPT // 2
FH = 32
THRESH = 1e-4


def _bcast16(vec, i):
    return vec.at[jnp.full((16,), i, jnp.int32)].get(mode="promise_in_bounds")


def _scale_to(dst, src, vals_v, j):
    for g in range(CH // 16):
        vv = vals_v[j, pl.ds(g * 16, 16)]
        for i in range(16):
            r = g * 16 + i
            fb = _bcast16(vv, i)
            v0, v1 = plsc.unpack(src[r], format=plsc.PackFormat.INTERLEAVED)
            dst[r, pl.ds(0, 16)] = v0 * fb
            dst[r, pl.ds(16, 16)] = v1 * fb


def _sc_layer(h, dr, dc, dv, filt):
    mesh = plsc.VectorSubcoreMesh(core_axis_name="c", subcore_axis_name="s")

    @functools.partial(
        pl.kernel,
        mesh=mesh,
        compiler_params=pltpu.CompilerParams(
            use_tc_tiling_on_sc=False, needs_layout_passes=False),
        out_type=jax.ShapeDtypeStruct((2, NPAD, FH), jnp.float32),
        scratch_types=[
            pltpu.VMEM((NCH, CH), jnp.int32),
            pltpu.VMEM((NCH, CH), jnp.int32),
            pltpu.VMEM((NCH, CH), jnp.float32),
            [pltpu.VMEM((CH, FH), jnp.bfloat16) for _ in range(NBUF)],
            [pltpu.VMEM((CH, FH), jnp.float32) for _ in range(NBUF)],
            pltpu.VMEM((HSTG, FH), jnp.float32),
            pltpu.VMEM((HSTG, FH), jnp.bfloat16),
            pltpu.VMEM((RPT,), jnp.float32),
            pltpu.VMEM_SHARED((NPAD, FH), jnp.bfloat16),
            pltpu.VMEM_SHARED((NPAD, FH), jnp.float32),
            pltpu.VMEM_SHARED((NPAD, FH), jnp.bfloat16),
            pltpu.VMEM_SHARED((NPAD, FH), jnp.float32),
            pltpu.SemaphoreType.DMA((NBUF,)),
            pltpu.SemaphoreType.DMA((NBUF,)),
        ],
    )
    def k(h_hbm, dr_hbm, dc_hbm, dv_hbm, f_hbm, out_hbm,
          cols_v, rows_v, vals_v, g_b, s_b, stg_v, stg_b, f_v,
          h_s, t_s, t_b, o_s, gsem, ssem):
        cid = lax.axis_index("c")
        sid = lax.axis_index("s")
        base = sid * RPT
        zrow = jnp.zeros((16,), jnp.float32)

        def zero_stg(i, _):
            stg_v[i, pl.ds(0, 16)] = zrow
            stg_v[i, pl.ds(16, 16)] = zrow
            return 0

        def zero_rings(i, _):
            for b in range(NBUF):
                s_b[b][i, pl.ds(0, 16)] = zrow
                s_b[b][i, pl.ds(16, 16)] = zrow
            return 0

        def phase(table, acc):
            lax.fori_loop(0, CH, zero_rings, 0)
            for b in range(NBUF):
                pltpu.async_copy(table.at[cols_v.at[b]], g_b[b], gsem.at[b])
                pltpu.async_copy(s_b[b], acc.at[rows_v.at[b]], ssem.at[b],
                                 add=True)

            def ring(gi, _):
                for b in range(NBUF):
                    j = gi * NBUF + b
                    pltpu.make_async_copy(
                        table.at[cols_v.at[j]], g_b[b], gsem.at[b]).wait()
                    pltpu.make_async_copy(
                        s_b[b], acc.at[rows_v.at[j]], ssem.at[b]).wait()
                    _scale_to(s_b[b], g_b[b], vals_v, j)
                    jn = jnp.minimum(j + NBUF, NCH - 1)
                    pltpu.async_copy(table.at[cols_v.at[jn]], g_b[b],
                                     gsem.at[b])
                    pltpu.async_copy(s_b[b], acc.at[rows_v.at[j]],
                                     ssem.at[b], add=True)
                return 0

            lax.fori_loop(0, NCH // NBUF, ring, 0)
            for b in range(NBUF):
                pltpu.make_async_copy(
                    table.at[cols_v.at[0]], g_b[b], gsem.at[b]).wait()
                pltpu.make_async_copy(
                    s_b[b], acc.at[rows_v.at[0]], ssem.at[b]).wait()

        lax.fori_loop(0, HSTG, zero_stg, 0)
        for hh in range(2):
            sl = pl.ds(base + hh * HSTG, HSTG)
            pltpu.sync_copy(stg_v, o_s.at[sl])
        for hh in range(2):
            sl = pl.ds(base + hh * HSTG, HSTG)
            pltpu.sync_copy(h_hbm.at[cid, sl], stg_b)
            pltpu.sync_copy(stg_b, h_s.at[sl])
        plsc.subcore_barrier()

        def k_body(k_i, _):
            lax.fori_loop(0, HSTG, zero_stg, 0)
            for hh in range(2):
                pltpu.sync_copy(stg_v, t_s.at[pl.ds(base + hh * HSTG, HSTG)])
            pltpu.sync_copy(dc_hbm.at[k_i, sid], cols_v)
            pltpu.sync_copy(dr_hbm.at[k_i, sid], rows_v)
            pltpu.sync_copy(dv_hbm.at[k_i, sid], vals_v)
            pltpu.sync_copy(f_hbm.at[k_i, sid], f_v)
            plsc.subcore_barrier()

            phase(h_s, t_s)
            plsc.subcore_barrier()

            for hh in range(2):
                sl = pl.ds(base + hh * HSTG, HSTG)
                pltpu.sync_copy(t_s.at[sl], stg_v)

                def shr_body(g, _):
                    fv = f_v[pl.ds(hh * HSTG + g * 16, 16)]
                    for i in range(16):
                        r = g * 16 + i
                        fb = _bcast16(fv, i)
                        ys = []
                        for half in range(2):
                            v = stg_v[r, pl.ds(half * 16, 16)]
                            a = jnp.maximum(jnp.abs(v) - THRESH, 0.0)
                            ys.append(jnp.sign(v) * a * fb)
                        stg_b[r] = plsc.pack(
                            ys[0], ys[1], format=plsc.PackFormat.INTERLEAVED)
                    return 0

                lax.fori_loop(0, HSTG // 16, shr_body, 0)
                pltpu.sync_copy(stg_b, t_b.at[sl])
            plsc.subcore_barrier()

            phase(t_b, o_s)
            plsc.subcore_barrier()
            return 0

        lax.fori_loop(0, NK, k_body, 0)

        for hh in range(2):
            sl = pl.ds(base + hh * HSTG, HSTG)
            pltpu.sync_copy(o_s.at[sl], stg_v)
            pltpu.sync_copy(stg_v, out_hbm.at[cid, sl])

    return k(h, dr, dc, dv, filt)


BN = 1280


def _mm1_body(x_ref, w_ref, o_ref):
    h = jnp.dot(x_ref[...], w_ref[...], preferred_element_type=jnp.float32)
    h = h.astype(jnp.bfloat16)
    o_ref[0] = h[:, :FH]
    o_ref[1] = h[:, FH:]


def _mm1(x, w):
    return pl.pallas_call(
        _mm1_body,
        grid=(NPAD // BN,),
        in_specs=[
            pl.BlockSpec((BN, IN_F), lambda i: (i, 0)),
            pl.BlockSpec((IN_F, HID), lambda i: (0, 0)),
        ],
        out_specs=pl.BlockSpec((2, BN, FH), lambda i: (0, i, 0)),
        out_shape=jax.ShapeDtypeStruct((2, NPAD, FH), jnp.bfloat16),
    )(x, w)


def _mm2_body(h_ref, w_ref, b_ref, o_ref):
    acc = jnp.dot(h_ref[0], w_ref[:FH, :], preferred_element_type=jnp.float32)
    acc += jnp.dot(h_ref[1], w_ref[FH:, :], preferred_element_type=jnp.float32)
    acc += b_ref[...]
    acc = acc.astype(jnp.bfloat16)
    o_ref[0] = acc[:, :FH]
    o_ref[1] = acc[:, FH:]


def _mm2(h, w, brow):
    return pl.pallas_call(
        _mm2_body,
        grid=(NPAD // BN,),
        in_specs=[
            pl.BlockSpec((2, BN, FH), lambda i: (0, i, 0)),
            pl.BlockSpec((HID, NCP), lambda i: (0, 0)),
            pl.BlockSpec((1, NCP), lambda i: (0, 0)),
        ],
        out_specs=pl.BlockSpec((2, BN, FH), lambda i: (0, i, 0)),
        out_shape=jax.ShapeDtypeStruct((2, NPAD, FH), jnp.bfloat16),
    )(h, w, brow)


FBN = 2000


def _final_body(h_ref, b_ref, o_ref):
    y = jnp.concatenate([h_ref[0], h_ref[1]], axis=1) + b_ref[...]
    col = lax.broadcasted_iota(jnp.int32, (FBN, NCP), 1)
    valid = col < NC
    neg = jnp.full_like(y, -jnp.inf)
    m = jnp.max(jnp.where(valid, y, neg), axis=1, keepdims=True)
    e = jnp.where(valid, jnp.exp(y - m), 0.0)
    s = jnp.sum(e, axis=1, keepdims=True)
    o_ref[...] = (y - m - jnp.log(s))[:, :NC]


def _final(h, b2p):
    return pl.pallas_call(
        _final_body,
        grid=(N // FBN,),
        in_specs=[
            pl.BlockSpec((2, FBN, FH), lambda i: (0, i, 0)),
            pl.BlockSpec((1, NCP), lambda i: (0, 0)),
        ],
        out_specs=pl.BlockSpec((FBN, NC), lambda i: (i, 0)),
        out_shape=jax.ShapeDtypeStruct((N, NC), jnp.float32),
    )(h, b2p)


def kernel(x, edge_index, W1, filt1, b1, W2, filt2, b2, d_rows, d_cols, d_vals):
    pad = ((0, 0), (0, 0), (0, EDG_TP - EDG_T))
    dr = jnp.pad(d_rows[1:RL].reshape(NK, NTILES, EDG_T), pad)
    dr = dr.reshape(NK, NTILES, NCH, CH)
    dc = jnp.pad(d_cols[1:RL].reshape(NK, NTILES, EDG_T), pad)
    dc = dc.reshape(NK, NTILES, NCH, CH)
    dv = jnp.pad(d_vals[1:RL].reshape(NK, NTILES, EDG_T), pad)
    dv = dv.reshape(NK, NTILES, NCH, CH)
    f1 = jnp.pad(filt1.reshape(RL, N)[1:RL], ((0, 0), (0, NPAD - N)))
    f1 = f1.reshape(NK, NTILES, RPT)
    f2 = jnp.pad(filt2.reshape(RL, N)[1:RL], ((0, 0), (0, NPAD - N)))
    f2 = f2.reshape(NK, NTILES, RPT)
    xp = jnp.pad(x, ((0, NPAD - N), (0, 0)))
    perm = []
    for c in range(2):
        for i in range(16):
            perm.extend((c * 32 + i, c * 32 + 16 + i))
    W1p = W1[:, jnp.array(perm, jnp.int32)]
    W2p = jnp.pad(W2, ((0, 0), (0, NCP - NC)))[:, jnp.array(perm, jnp.int32)]
    bias2 = (b1 @ W2p).reshape(1, NCP)
    b2p = jnp.pad(b2, (0, NCP - NC)).reshape(1, NCP)

    h1 = _mm1(xp, W1p)
    o1 = _sc_layer(h1, dr, dc, dv, f1)
    h2 = _mm2(o1, W2p, bias2)
    o2 = _sc_layer(h2, dr, dc, dv, f2)
    o2c = lax.slice(o2, (0, 0, 0), (2, N, FH))
    return _final(o2c, b2p)


# speedup vs baseline: 16.4176x; 1.1883x over previous
import functools

import jax
import jax.numpy as jnp
from jax import lax
from jax.experimental import pallas as pl
from jax.experimental.pallas import tpu as pltpu
from jax.experimental.pallas import tpu_sc as plsc

N = 10000
NPAD = 10240
IN_F = 128
HID = 64
NC = 40
NCP = 64
RL = 4
NK = 3
NNZ = 160000
NTILES = 16
EDG_T = NNZ // NTILES
CH = 80
NCH = EDG_T // CH
NBUF = 5
RPT = NPAD // NTILES
HSTG = RPT // 2
FH = 32
THRESH = 1e-4


def _bcast16(vec, i):
    return vec.at[jnp.full((16,), i, jnp.int32)].get(mode="promise_in_bounds")


def _scale_to(dst, src, vals_v, j):
    for g in range(CH // 16):
        vv = vals_v[j, pl.ds(g * 16, 16)]
        for i in range(16):
            r = g * 16 + i
            fb = _bcast16(vv, i)
            v0, v1 = plsc.unpack(src[r], format=plsc.PackFormat.INTERLEAVED)
            dst[r, pl.ds(0, 16)] = v0 * fb
            dst[r, pl.ds(16, 16)] = v1 * fb


def _sc_layer(h, dr, dc, dv, filt):
    mesh = plsc.VectorSubcoreMesh(core_axis_name="c", subcore_axis_name="s")

    @functools.partial(
        pl.kernel,
        mesh=mesh,
        compiler_params=pltpu.CompilerParams(
            use_tc_tiling_on_sc=False, needs_layout_passes=False),
        out_type=jax.ShapeDtypeStruct((2, NPAD, FH), jnp.float32),
        scratch_types=[
            pltpu.VMEM((NCH, CH), jnp.int32),
            pltpu.VMEM((NCH, CH), jnp.int32),
            pltpu.VMEM((NCH, CH), jnp.float32),
            [pltpu.VMEM((CH, FH), jnp.bfloat16) for _ in range(NBUF)],
            [pltpu.VMEM((CH, FH), jnp.float32) for _ in range(NBUF)],
            pltpu.VMEM((HSTG, FH), jnp.float32),
            pltpu.VMEM((HSTG, FH), jnp.bfloat16),
            pltpu.VMEM((RPT,), jnp.float32),
            pltpu.VMEM_SHARED((NPAD, FH), jnp.bfloat16),
            pltpu.VMEM_SHARED((NPAD, FH), jnp.float32),
            pltpu.VMEM_SHARED((NPAD, FH), jnp.bfloat16),
            pltpu.VMEM_SHARED((NPAD, FH), jnp.float32),
            pltpu.SemaphoreType.DMA((NBUF,)),
            pltpu.SemaphoreType.DMA((NBUF,)),
        ],
    )
    def k(h_hbm, dr_hbm, dc_hbm, dv_hbm, f_hbm, out_hbm,
          cols_v, rows_v, vals_v, g_b, s_b, stg_v, stg_b, f_v,
          h_s, t_s, t_b, o_s, gsem, ssem):
        cid = lax.axis_index("c")
        sid = lax.axis_index("s")
        base = sid * RPT
        zrow = jnp.zeros((16,), jnp.float32)

        def zero_stg(i, _):
            stg_v[i, pl.ds(0, 16)] = zrow
            stg_v[i, pl.ds(16, 16)] = zrow
            return 0

        def zero_rings(i, _):
            for b in range(NBUF):
                s_b[b][i, pl.ds(0, 16)] = zrow
                s_b[b][i, pl.ds(16, 16)] = zrow
            return 0

        def phase(table, acc):
            lax.fori_loop(0, CH, zero_rings, 0)
            for b in range(NBUF):
                pltpu.async_copy(table.at[cols_v.at[b]], g_b[b], gsem.at[b])
                pltpu.async_copy(s_b[b], acc.at[rows_v.at[b]], ssem.at[b],
                                 add=True)

            def ring(gi, _):
                for b in range(NBUF):
                    j = gi * NBUF + b
                    pltpu.make_async_copy(
                        table.at[cols_v.at[j]], g_b[b], gsem.at[b]).wait()
                    pltpu.make_async_copy(
                        s_b[b], acc.at[rows_v.at[j]], ssem.at[b]).wait()
                    _scale_to(s_b[b], g_b[b], vals_v, j)
                    jn = jnp.minimum(j + NBUF, NCH - 1)
                    pltpu.async_copy(table.at[cols_v.at[jn]], g_b[b],
                                     gsem.at[b])
                    pltpu.async_copy(s_b[b], acc.at[rows_v.at[j]],
                                     ssem.at[b], add=True)
                return 0

            lax.fori_loop(0, NCH // NBUF, ring, 0)
            for b in range(NBUF):
                pltpu.make_async_copy(
                    table.at[cols_v.at[0]], g_b[b], gsem.at[b]).wait()
                pltpu.make_async_copy(
                    s_b[b], acc.at[rows_v.at[0]], ssem.at[b]).wait()

        lax.fori_loop(0, HSTG, zero_stg, 0)
        for hh in range(2):
            sl = pl.ds(base + hh * HSTG, HSTG)
            pltpu.sync_copy(stg_v, o_s.at[sl])
        for hh in range(2):
            sl = pl.ds(base + hh * HSTG, HSTG)
            pltpu.sync_copy(h_hbm.at[cid, sl], stg_b)
            pltpu.sync_copy(stg_b, h_s.at[sl])
        plsc.subcore_barrier()

        def k_body(k_i, _):
            lax.fori_loop(0, HSTG, zero_stg, 0)
            for hh in range(2):
                pltpu.sync_copy(stg_v, t_s.at[pl.ds(base + hh * HSTG, HSTG)])
            pltpu.sync_copy(dc_hbm.at[k_i + 1, sid], cols_v)
            pltpu.sync_copy(dr_hbm.at[k_i + 1, sid], rows_v)
            pltpu.sync_copy(dv_hbm.at[k_i + 1, sid], vals_v)
            pltpu.sync_copy(f_hbm.at[k_i, sid], f_v)
            plsc.subcore_barrier()

            phase(h_s, t_s)
            plsc.subcore_barrier()

            for hh in range(2):
                sl = pl.ds(base + hh * HSTG, HSTG)
                pltpu.sync_copy(t_s.at[sl], stg_v)

                def shr_body(g, _):
                    fv = f_v[pl.ds(hh * HSTG + g * 16, 16)]
                    for i in range(16):
                        r = g * 16 + i
                        fb = _bcast16(fv, i)
                        ys = []
                        for half in range(2):
                            v = stg_v[r, pl.ds(half * 16, 16)]
                            a = jnp.maximum(jnp.abs(v) - THRESH, 0.0)
                            ys.append(jnp.sign(v) * a * fb)
                        stg_b[r] = plsc.pack(
                            ys[0], ys[1], format=plsc.PackFormat.INTERLEAVED)
                    return 0

                lax.fori_loop(0, HSTG // 16, shr_body, 0)
                pltpu.sync_copy(stg_b, t_b.at[sl])
            plsc.subcore_barrier()

            phase(t_b, o_s)
            plsc.subcore_barrier()
            return 0

        lax.fori_loop(0, NK, k_body, 0)

        for hh in range(2):
            sl = pl.ds(base + hh * HSTG, HSTG)
            pltpu.sync_copy(o_s.at[sl], stg_v)
            pltpu.sync_copy(stg_v, out_hbm.at[cid, sl])

    return k(h, dr, dc, dv, filt)


BN = 1280


def _mm1_body(x_ref, w_ref, o_ref):
    h = jnp.dot(x_ref[...], w_ref[...], preferred_element_type=jnp.float32)
    h = h.astype(jnp.bfloat16)
    o_ref[0] = h[:, :FH]
    o_ref[1] = h[:, FH:]


def _mm1(x, w):
    return pl.pallas_call(
        _mm1_body,
        grid=(NPAD // BN,),
        in_specs=[
            pl.BlockSpec((BN, IN_F), lambda i: (i, 0)),
            pl.BlockSpec((IN_F, HID), lambda i: (0, 0)),
        ],
        out_specs=pl.BlockSpec((2, BN, FH), lambda i: (0, i, 0)),
        out_shape=jax.ShapeDtypeStruct((2, NPAD, FH), jnp.bfloat16),
    )(x, w)


def _mm2_body(h_ref, w_ref, b_ref, o_ref):
    acc = jnp.dot(h_ref[0], w_ref[:FH, :], preferred_element_type=jnp.float32)
    acc += jnp.dot(h_ref[1], w_ref[FH:, :], preferred_element_type=jnp.float32)
    acc += b_ref[...]
    acc = acc.astype(jnp.bfloat16)
    o_ref[0] = acc[:, :FH]
    o_ref[1] = acc[:, FH:]


def _mm2(h, w, brow):
    return pl.pallas_call(
        _mm2_body,
        grid=(NPAD // BN,),
        in_specs=[
            pl.BlockSpec((2, BN, FH), lambda i: (0, i, 0)),
            pl.BlockSpec((HID, NCP), lambda i: (0, 0)),
            pl.BlockSpec((1, NCP), lambda i: (0, 0)),
        ],
        out_specs=pl.BlockSpec((2, BN, FH), lambda i: (0, i, 0)),
        out_shape=jax.ShapeDtypeStruct((2, NPAD, FH), jnp.bfloat16),
    )(h, w, brow)


FBN = 2000


def _final_body(h_ref, b_ref, o_ref):
    y = jnp.concatenate([h_ref[0], h_ref[1]], axis=1) + b_ref[...]
    col = lax.broadcasted_iota(jnp.int32, (FBN, NCP), 1)
    valid = col < NC
    neg = jnp.full_like(y, -jnp.inf)
    m = jnp.max(jnp.where(valid, y, neg), axis=1, keepdims=True)
    e = jnp.where(valid, jnp.exp(y - m), 0.0)
    s = jnp.sum(e, axis=1, keepdims=True)
    o_ref[...] = (y - m - jnp.log(s))[:, :NC]


def _final(h, b2p):
    return pl.pallas_call(
        _final_body,
        grid=(N // FBN,),
        in_specs=[
            pl.BlockSpec((2, FBN, FH), lambda i: (0, i, 0)),
            pl.BlockSpec((1, NCP), lambda i: (0, 0)),
        ],
        out_specs=pl.BlockSpec((FBN, NC), lambda i: (i, 0)),
        out_shape=jax.ShapeDtypeStruct((N, NC), jnp.float32),
    )(h, b2p)


def kernel(x, edge_index, W1, filt1, b1, W2, filt2, b2, d_rows, d_cols, d_vals):
    dr = d_rows.reshape(RL, NTILES, NCH, CH)
    dc = d_cols.reshape(RL, NTILES, NCH, CH)
    dv = d_vals.reshape(RL, NTILES, NCH, CH)
    f1 = jnp.pad(filt1.reshape(RL, N)[1:RL], ((0, 0), (0, NPAD - N)))
    f1 = f1.reshape(NK, NTILES, RPT)
    f2 = jnp.pad(filt2.reshape(RL, N)[1:RL], ((0, 0), (0, NPAD - N)))
    f2 = f2.reshape(NK, NTILES, RPT)
    perm = []
    for c in range(2):
        for i in range(16):
            perm.extend((c * 32 + i, c * 32 + 16 + i))
    W1p = W1[:, jnp.array(perm, jnp.int32)]
    W2p = jnp.pad(W2, ((0, 0), (0, NCP - NC)))[:, jnp.array(perm, jnp.int32)]
    bias2 = (b1 @ W2p).reshape(1, NCP)
    b2p = jnp.pad(b2, (0, NCP - NC)).reshape(1, NCP)

    h1 = _mm1(x, W1p)
    o1 = _sc_layer(h1, dr, dc, dv, f1)
    h2 = _mm2(o1, W2p, bias2)
    o2 = _sc_layer(h2, dr, dc, dv, f2)
    return _final(o2, b2p)
